# baseline (speedup 1.0000x reference)
import functools

import jax
import jax.numpy as jnp
from jax import lax
from jax.experimental import pallas as pl
from jax.experimental.pallas import tpu as pltpu
from jax.experimental.pallas import tpu_sc as plsc

N = 10000
NP = 10240
E = 320000
EP = 327680
NF = 16
NCAT = 1000
EMB = 8
EMBP = 16
IN = NF * EMB
INP = NF * EMBP
HID = 128
OUT = 64

NC = 2
NS = 16
NW = NC * NS
LANES = 16

ROWS_PER_TILE = NP // NS

_MESH = plsc.VectorSubcoreMesh(
    core_axis_name="c", subcore_axis_name="s", num_cores=NC, num_subcores=NS)
_SC_PARAMS = pltpu.CompilerParams(use_tc_tiling_on_sc=False)


def _fill(ref, nrows, ncols, value):
    nv = ncols // LANES
    v = jnp.full((LANES,), value, jnp.float32)

    def body(i, carry):
        r = i // nv
        k = i % nv
        ref[r, pl.ds(k * LANES, LANES)] = v
        return carry

    lax.fori_loop(0, nrows * nv, body, 0)


EMB_CHUNK = 128
EMB_CHUNKS = (NP * NF // NW) // EMB_CHUNK
DEG_CHUNK = 128
DEG_CHUNKS = (EP // NW) // DEG_CHUNK


def _emb_deg_body(fidx_hbm, emb_hbm, ecol_hbm, xpad_out, deg_out,
                  zb16, ones_v, colidx, fidxb, g0, g1, dacc, sem0, sem1,
                  ps0, ps1):
    c = lax.axis_index("c")
    s = lax.axis_index("s")
    w = c * NS + s

    _fill(zb16, ROWS_PER_TILE, EMBP, 0.0)
    pltpu.sync_copy(zb16, dacc.at[pl.ds(s * ROWS_PER_TILE, ROWS_PER_TILE)])
    _fill(ones_v, DEG_CHUNK, EMBP, 1.0)
    pltpu.sync_copy(ecol_hbm.at[pl.ds(w * DEG_CHUNKS, DEG_CHUNKS)], colidx)
    pltpu.sync_copy(fidx_hbm.at[pl.ds(w * EMB_CHUNKS, EMB_CHUNKS)], fidxb)
    plsc.subcore_barrier()

    def dbody(i2, carry):
        i = 2 * i2
        pltpu.async_copy(ones_v, dacc.at[colidx.at[i]], ps0, add=True)
        pltpu.async_copy(ones_v, dacc.at[colidx.at[i + 1]], ps1, add=True)
        pltpu.make_async_copy(ones_v, dacc.at[colidx.at[0]], ps0).wait()
        pltpu.make_async_copy(ones_v, dacc.at[colidx.at[0]], ps1).wait()
        return carry

    lax.fori_loop(0, DEG_CHUNKS // 2, dbody, 0)

    lbase = w * (NP * NF // NW)

    def _fire(i, gb, sm):
        pltpu.async_copy(emb_hbm.at[fidxb.at[i]], gb, sm)

    def _gwait(gb, sm):
        pltpu.make_async_copy(emb_hbm.at[fidxb.at[0]], gb, sm).wait()

    def _put(i, gb, sm):
        pltpu.async_copy(
            gb, xpad_out.at[pl.ds(lbase + i * EMB_CHUNK, EMB_CHUNK)], sm)

    def _pwait(gb, sm):
        pltpu.make_async_copy(
            gb, xpad_out.at[pl.ds(lbase, EMB_CHUNK)], sm).wait()

    _fire(0, g0, sem0)
    _fire(1, g1, sem1)

    def ebody(i2, carry):
        i = 2 * i2
        _gwait(g0, sem0)
        _put(i, g0, ps0)
        _gwait(g1, sem1)
        _put(i + 1, g1, ps1)

        @pl.when(i + 2 < EMB_CHUNKS)
        def _():
            _pwait(g0, ps0)
            _fire(i + 2, g0, sem0)

        @pl.when(i + 3 < EMB_CHUNKS)
        def _():
            _pwait(g1, ps1)
            _fire(i + 3, g1, sem1)

        return carry

    lax.fori_loop(0, EMB_CHUNKS // 2, ebody, 0)
    _pwait(g0, ps0)
    _pwait(g1, ps1)

    plsc.subcore_barrier()
    pltpu.sync_copy(dacc.at[pl.ds(s * ROWS_PER_TILE, ROWS_PER_TILE)], zb16)
    pltpu.sync_copy(zb16, deg_out.at[c, pl.ds(s * ROWS_PER_TILE, ROWS_PER_TILE)])


_emb_deg_kernel = pl.kernel(
    _emb_deg_body,
    out_type=(
        jax.ShapeDtypeStruct((NP * NF, EMBP), jnp.float32),
        jax.ShapeDtypeStruct((NC, NP, EMBP), jnp.float32),
    ),
    mesh=_MESH,
    compiler_params=_SC_PARAMS,
    scratch_types=[
        pltpu.VMEM((ROWS_PER_TILE, EMBP), jnp.float32),
        pltpu.VMEM((DEG_CHUNK, EMBP), jnp.float32),
        pltpu.VMEM((DEG_CHUNKS, DEG_CHUNK), jnp.int32),
        pltpu.VMEM((EMB_CHUNKS, EMB_CHUNK), jnp.int32),
        pltpu.VMEM((EMB_CHUNK, EMBP), jnp.float32),
        pltpu.VMEM((EMB_CHUNK, EMBP), jnp.float32),
        pltpu.VMEM_SHARED((NP, EMBP), jnp.float32),
        pltpu.SemaphoreType.DMA,
        pltpu.SemaphoreType.DMA,
        pltpu.SemaphoreType.DMA,
        pltpu.SemaphoreType.DMA,
    ],
)


EDGE_CHUNK = 128
EDGE_CHUNKS = (EP // NS) // EDGE_CHUNK


def _scatter_body(w, yh_hbm, erow_hbm, ecol_hbm, sh_out,
                  rowidx, colidx, g0, g1, acc, sem0, sem1):
    c = lax.axis_index("c")
    s = lax.axis_index("s")

    _fill(g0, EDGE_CHUNK, w, 0.0)

    def zbody(r, carry):
        pltpu.sync_copy(
            g0, acc.at[pl.ds(s * ROWS_PER_TILE + r * EDGE_CHUNK, EDGE_CHUNK)])
        return carry

    lax.fori_loop(0, ROWS_PER_TILE // EDGE_CHUNK, zbody, 0)
    pltpu.sync_copy(erow_hbm.at[pl.ds(s * EDGE_CHUNKS, EDGE_CHUNKS)], rowidx)
    pltpu.sync_copy(ecol_hbm.at[pl.ds(s * EDGE_CHUNKS, EDGE_CHUNKS)], colidx)
    roff = c * NP

    def adj(k, carry):
        r = k // (EDGE_CHUNK // LANES)
        j = k % (EDGE_CHUNK // LANES)
        rowidx[r, pl.ds(j * LANES, LANES)] = (
            rowidx[r, pl.ds(j * LANES, LANES)] + roff)
        return carry

    @pl.when(c > 0)
    def _():
        lax.fori_loop(0, EDGE_CHUNKS * (EDGE_CHUNK // LANES), adj, 0)

    plsc.subcore_barrier()

    def _fire(i, gb, sm):
        pltpu.async_copy(yh_hbm.at[rowidx.at[i]], gb, sm)

    def _gwait(gb, sm):
        pltpu.make_async_copy(yh_hbm.at[rowidx.at[0]], gb, sm).wait()

    def _scat(i, gb):
        pltpu.sync_copy(gb, acc.at[colidx.at[i]], add=True)

    _fire(0, g0, sem0)

    def body(i2, carry):
        i = 2 * i2
        _fire(i + 1, g1, sem1)
        _gwait(g0, sem0)
        _scat(i, g0)

        @pl.when(i + 2 < EDGE_CHUNKS)
        def _():
            _fire(i + 2, g0, sem0)

        _gwait(g1, sem1)
        _scat(i + 1, g1)
        return carry

    lax.fori_loop(0, EDGE_CHUNKS // 2, body, 0)

    plsc.subcore_barrier()

    def obody(r, carry):
        base = s * ROWS_PER_TILE + r * EDGE_CHUNK
        pltpu.sync_copy(acc.at[pl.ds(base, EDGE_CHUNK)], g0)
        pltpu.sync_copy(g0, sh_out.at[c, pl.ds(base, EDGE_CHUNK)])
        return carry

    lax.fori_loop(0, ROWS_PER_TILE // EDGE_CHUNK, obody, 0)


@functools.cache
def _make_scatter_kernel(w):
    return pl.kernel(
        functools.partial(_scatter_body, w),
        out_type=jax.ShapeDtypeStruct((NC, NP, w), jnp.float32),
        mesh=_MESH,
        compiler_params=_SC_PARAMS,
        scratch_types=[
            pltpu.VMEM((EDGE_CHUNKS, EDGE_CHUNK), jnp.int32),
            pltpu.VMEM((EDGE_CHUNKS, EDGE_CHUNK), jnp.int32),
            pltpu.VMEM((EDGE_CHUNK, w), jnp.float32),
            pltpu.VMEM((EDGE_CHUNK, w), jnp.float32),
            pltpu.VMEM_SHARED((NP, w), jnp.float32),
            pltpu.SemaphoreType.DMA,
            pltpu.SemaphoreType.DMA,
        ],
    )


ROW_BLK = 2048


def _dis_block(deg_ref):
    cnt = deg_ref[0, :, :1] + deg_ref[1, :, :1]
    return lax.rsqrt(cnt + 1.0)


def _tc_b_body(x_ref, w_ref, deg_ref, y_ref):
    dis = _dis_block(deg_ref)
    y = jnp.dot(x_ref[...], w_ref[...], preferred_element_type=jnp.float32)
    y = y * dis
    y_ref[0] = y[:, :HID // 2]
    y_ref[1] = y[:, HID // 2:]


def _tc_d_body(s_ref, y_ref, deg_ref, b1_ref, w2_ref, o_ref):
    dis = _dis_block(deg_ref)
    b1 = b1_ref[...]
    t0 = dis * (s_ref[0] + y_ref[0]) + b1[:, :HID // 2]
    t1 = dis * (s_ref[1] + y_ref[1]) + b1[:, HID // 2:]
    h = jnp.maximum(jnp.concatenate([t0, t1], axis=1), 0.0)
    y2 = jnp.dot(h, w2_ref[...], preferred_element_type=jnp.float32) * dis
    o_ref[0] = y2[:, :OUT // 2]
    o_ref[1] = y2[:, OUT // 2:]


def _tc_f_body(s_ref, y_ref, deg_ref, b2_ref, o_ref):
    dis = _dis_block(deg_ref)
    full = jnp.concatenate(
        [s_ref[0] + y_ref[0], s_ref[1] + y_ref[1]], axis=1)
    o_ref[...] = dis * full + b2_ref[...]


def _halves_spec(w):
    return pl.BlockSpec((NC, ROW_BLK, w), lambda i: (0, i, 0))


_DEG_SPEC = pl.BlockSpec((NC, ROW_BLK, EMBP), lambda i: (0, i, 0))
_GRID = (NP // ROW_BLK,)


def _tc_b(xpad, w1p, deg):
    return pl.pallas_call(
        _tc_b_body,
        grid=_GRID,
        in_specs=[
            pl.BlockSpec((ROW_BLK, INP), lambda i: (i, 0)),
            pl.BlockSpec((INP, HID), lambda i: (0, 0)),
            _DEG_SPEC,
        ],
        out_specs=_halves_spec(HID // 2),
        out_shape=jax.ShapeDtypeStruct((NC, NP, HID // 2), jnp.float32),
    )(xpad, w1p, deg)


def _tc_d(s1h, y1h, deg, b1, w2):
    return pl.pallas_call(
        _tc_d_body,
        grid=_GRID,
        in_specs=[
            _halves_spec(HID // 2),
            _halves_spec(HID // 2),
            _DEG_SPEC,
            pl.BlockSpec((1, HID), lambda i: (0, 0)),
            pl.BlockSpec((HID, OUT), lambda i: (0, 0)),
        ],
        out_specs=_halves_spec(OUT // 2),
        out_shape=jax.ShapeDtypeStruct((NC, NP, OUT // 2), jnp.float32),
    )(s1h, y1h, deg, b1, w2)


def _tc_f(s2h, y2h, deg, b2):
    return pl.pallas_call(
        _tc_f_body,
        grid=_GRID,
        in_specs=[
            _halves_spec(OUT // 2),
            _halves_spec(OUT // 2),
            _DEG_SPEC,
            pl.BlockSpec((1, OUT), lambda i: (0, 0)),
        ],
        out_specs=pl.BlockSpec((ROW_BLK, OUT), lambda i: (i, 0)),
        out_shape=jax.ShapeDtypeStruct((N, OUT), jnp.float32),
    )(s2h, y2h, deg, b2)


def kernel(x_cat, edge_index, emb_tables, W1, b1, W2, b2):
    xcat_pad = jnp.pad(x_cat, ((0, NP - N), (0, 0)))
    fidx = (xcat_pad
            + jnp.arange(NF, dtype=jnp.int32)[None, :] * NCAT).reshape(-1)
    emb_pad = jnp.pad(emb_tables.reshape(NF * NCAT, EMB),
                      ((0, 0), (0, EMBP - EMB)))
    w1p = jnp.zeros((NF, EMBP, HID), jnp.float32).at[:, :EMB, :].set(
        W1.reshape(NF, EMB, HID)).reshape(INP, HID)
    ei = jnp.pad(edge_index, ((0, 0), (0, EP - E)), constant_values=NP - 1)
    erow = ei[0].reshape(EP // EDGE_CHUNK, EDGE_CHUNK)
    ecol = ei[1].reshape(EP // EDGE_CHUNK, EDGE_CHUNK)
    fidx = fidx.reshape((NP * NF) // EMB_CHUNK, EMB_CHUNK)

    xpad, deg = _emb_deg_kernel(fidx, emb_pad, ecol)
    xpad = xpad.reshape(NP, INP)

    y1h = _tc_b(xpad, w1p, deg)
    s1h = _make_scatter_kernel(HID // 2)(
        y1h.reshape(NC * NP, HID // 2), erow, ecol)
    y2h = _tc_d(s1h, y1h, deg, b1.reshape(1, HID), W2)
    s2h = _make_scatter_kernel(OUT // 2)(
        y2h.reshape(NC * NP, OUT // 2), erow, ecol)
    return _tc_f(s2h, y2h, deg, b2.reshape(1, OUT))


# speedup vs baseline: 1.6677x; 1.6677x over previous
import functools

import jax
import jax.numpy as jnp
from jax import lax
from jax.experimental import pallas as pl
from jax.experimental.pallas import tpu as pltpu
from jax.experimental.pallas import tpu_sc as plsc

N = 10000
NP = 10240
E = 320000
EP = 327680
NF = 16
NCAT = 1000
EMB = 8
EMBP = 16
IN = NF * EMB
INP = NF * EMBP
HID = 128
OUT = 64

NC = 2
NS = 16
NW = NC * NS
LANES = 16

ROWS_PER_TILE = NP // NS

_MESH = plsc.VectorSubcoreMesh(
    core_axis_name="c", subcore_axis_name="s", num_cores=NC, num_subcores=NS)
_SC_PARAMS = pltpu.CompilerParams(use_tc_tiling_on_sc=False)


def _fill(ref, nrows, ncols, value):
    nv = ncols // LANES
    v = jnp.full((LANES,), value, jnp.float32)

    def body(i, carry):
        r = i // nv
        k = i % nv
        ref[r, pl.ds(k * LANES, LANES)] = v
        return carry

    lax.fori_loop(0, nrows * nv, body, 0)


EMB_CHUNK = 128
EMB_CHUNKS = (NP * NF // NW) // EMB_CHUNK
DEG_CHUNK = 128
DEG_CHUNKS = (EP // NW) // DEG_CHUNK


def _emb_deg_body(fidx_hbm, emb_hbm, ecol_hbm, xpad_out, deg_out,
                  zb16, ones_v, colidx, fidxb, g0, g1, dacc, sem0, sem1,
                  ps0, ps1):
    c = lax.axis_index("c")
    s = lax.axis_index("s")
    w = c * NS + s

    _fill(zb16, ROWS_PER_TILE, EMBP, 0.0)
    pltpu.sync_copy(zb16, dacc.at[pl.ds(s * ROWS_PER_TILE, ROWS_PER_TILE)])
    _fill(ones_v, DEG_CHUNK, EMBP, 1.0)
    pltpu.sync_copy(ecol_hbm.at[pl.ds(w * DEG_CHUNKS, DEG_CHUNKS)], colidx)
    pltpu.sync_copy(fidx_hbm.at[pl.ds(w * EMB_CHUNKS, EMB_CHUNKS)], fidxb)
    plsc.subcore_barrier()

    def dbody(i2, carry):
        i = 2 * i2
        pltpu.async_copy(ones_v, dacc.at[colidx.at[i]], ps0, add=True)
        pltpu.async_copy(ones_v, dacc.at[colidx.at[i + 1]], ps1, add=True)
        pltpu.make_async_copy(ones_v, dacc.at[colidx.at[0]], ps0).wait()
        pltpu.make_async_copy(ones_v, dacc.at[colidx.at[0]], ps1).wait()
        return carry

    lax.fori_loop(0, DEG_CHUNKS // 2, dbody, 0)

    lbase = w * (NP * NF // NW)

    def _fire(i, gb, sm):
        pltpu.async_copy(emb_hbm.at[fidxb.at[i]], gb, sm)

    def _gwait(gb, sm):
        pltpu.make_async_copy(emb_hbm.at[fidxb.at[0]], gb, sm).wait()

    def _put(i, gb, sm):
        pltpu.async_copy(
            gb, xpad_out.at[pl.ds(lbase + i * EMB_CHUNK, EMB_CHUNK)], sm)

    def _pwait(gb, sm):
        pltpu.make_async_copy(
            gb, xpad_out.at[pl.ds(lbase, EMB_CHUNK)], sm).wait()

    _fire(0, g0, sem0)
    _fire(1, g1, sem1)

    def ebody(i2, carry):
        i = 2 * i2
        _gwait(g0, sem0)
        _put(i, g0, ps0)
        _gwait(g1, sem1)
        _put(i + 1, g1, ps1)

        @pl.when(i + 2 < EMB_CHUNKS)
        def _():
            _pwait(g0, ps0)
            _fire(i + 2, g0, sem0)

        @pl.when(i + 3 < EMB_CHUNKS)
        def _():
            _pwait(g1, ps1)
            _fire(i + 3, g1, sem1)

        return carry

    lax.fori_loop(0, EMB_CHUNKS // 2, ebody, 0)
    _pwait(g0, ps0)
    _pwait(g1, ps1)

    plsc.subcore_barrier()
    pltpu.sync_copy(dacc.at[pl.ds(s * ROWS_PER_TILE, ROWS_PER_TILE)], zb16)
    pltpu.sync_copy(zb16, deg_out.at[c, pl.ds(s * ROWS_PER_TILE, ROWS_PER_TILE)])


_emb_deg_kernel = pl.kernel(
    _emb_deg_body,
    out_type=(
        jax.ShapeDtypeStruct((NP * NF, EMBP), jnp.float32),
        jax.ShapeDtypeStruct((NC, NP, EMBP), jnp.float32),
    ),
    mesh=_MESH,
    compiler_params=_SC_PARAMS,
    scratch_types=[
        pltpu.VMEM((ROWS_PER_TILE, EMBP), jnp.float32),
        pltpu.VMEM((DEG_CHUNK, EMBP), jnp.float32),
        pltpu.VMEM((DEG_CHUNKS, DEG_CHUNK), jnp.int32),
        pltpu.VMEM((EMB_CHUNKS, EMB_CHUNK), jnp.int32),
        pltpu.VMEM((EMB_CHUNK, EMBP), jnp.float32),
        pltpu.VMEM((EMB_CHUNK, EMBP), jnp.float32),
        pltpu.VMEM_SHARED((NP, EMBP), jnp.float32),
        pltpu.SemaphoreType.DMA,
        pltpu.SemaphoreType.DMA,
        pltpu.SemaphoreType.DMA,
        pltpu.SemaphoreType.DMA,
    ],
)


EDGE_CHUNK = 128
EDGE_CHUNKS = (EP // NS) // EDGE_CHUNK


def _scatter_body(w, yh_hbm, erow_hbm, ecol_hbm, sh_out,
                  rowidx, colidx, g0, g1, acc, sem0, sem1):
    c = lax.axis_index("c")
    s = lax.axis_index("s")

    _fill(g0, EDGE_CHUNK, w, 0.0)

    def zbody(r, carry):
        pltpu.sync_copy(
            g0, acc.at[pl.ds(s * ROWS_PER_TILE + r * EDGE_CHUNK, EDGE_CHUNK)])
        return carry

    lax.fori_loop(0, ROWS_PER_TILE // EDGE_CHUNK, zbody, 0)
    pltpu.sync_copy(erow_hbm.at[pl.ds(s * EDGE_CHUNKS, EDGE_CHUNKS)], rowidx)
    pltpu.sync_copy(ecol_hbm.at[pl.ds(s * EDGE_CHUNKS, EDGE_CHUNKS)], colidx)
    roff = c * NP

    def adj(k, carry):
        r = k // (EDGE_CHUNK // LANES)
        j = k % (EDGE_CHUNK // LANES)
        rowidx[r, pl.ds(j * LANES, LANES)] = (
            rowidx[r, pl.ds(j * LANES, LANES)] + roff)
        return carry

    @pl.when(c > 0)
    def _():
        lax.fori_loop(0, EDGE_CHUNKS * (EDGE_CHUNK // LANES), adj, 0)

    plsc.subcore_barrier()

    def _fire(i, gb, sm):
        pltpu.async_copy(yh_hbm.at[rowidx.at[i]], gb, sm)

    def _gwait(gb, sm):
        pltpu.make_async_copy(yh_hbm.at[rowidx.at[0]], gb, sm).wait()

    def _scat(i, gb):
        pltpu.sync_copy(gb, acc.at[colidx.at[i]], add=True)

    _fire(0, g0, sem0)

    def body(i2, carry):
        i = 2 * i2
        _fire(i + 1, g1, sem1)
        _gwait(g0, sem0)
        _scat(i, g0)

        @pl.when(i + 2 < EDGE_CHUNKS)
        def _():
            _fire(i + 2, g0, sem0)

        _gwait(g1, sem1)
        _scat(i + 1, g1)
        return carry

    lax.fori_loop(0, EDGE_CHUNKS // 2, body, 0)

    plsc.subcore_barrier()

    def obody(r, carry):
        base = s * ROWS_PER_TILE + r * EDGE_CHUNK
        pltpu.sync_copy(acc.at[pl.ds(base, EDGE_CHUNK)], g0)
        pltpu.sync_copy(g0, sh_out.at[c, pl.ds(base, EDGE_CHUNK)])
        return carry

    lax.fori_loop(0, ROWS_PER_TILE // EDGE_CHUNK, obody, 0)


@functools.cache
def _make_scatter_kernel(w):
    return pl.kernel(
        functools.partial(_scatter_body, w),
        out_type=jax.ShapeDtypeStruct((NC, NP, w), jnp.float32),
        mesh=_MESH,
        compiler_params=_SC_PARAMS,
        scratch_types=[
            pltpu.VMEM((EDGE_CHUNKS, EDGE_CHUNK), jnp.int32),
            pltpu.VMEM((EDGE_CHUNKS, EDGE_CHUNK), jnp.int32),
            pltpu.VMEM((EDGE_CHUNK, w), jnp.float32),
            pltpu.VMEM((EDGE_CHUNK, w), jnp.float32),
            pltpu.VMEM_SHARED((NP, w), jnp.float32),
            pltpu.SemaphoreType.DMA,
            pltpu.SemaphoreType.DMA,
        ],
    )


ROW_BLK = 2048


def _dis_block(deg_ref):
    cnt = deg_ref[0, :, :1] + deg_ref[1, :, :1]
    return lax.rsqrt(cnt + 1.0)


def _tc_b_body(x_ref, w_ref, deg_ref, y_ref):
    dis = _dis_block(deg_ref)
    y = jnp.dot(x_ref[...], w_ref[...], preferred_element_type=jnp.float32)
    y = y * dis
    y_ref[0] = y[:, :HID // 2]
    y_ref[1] = y[:, HID // 2:]


def _tc_d_body(s_ref, y_ref, deg_ref, b1_ref, w2_ref, o_ref):
    dis = _dis_block(deg_ref)
    b1 = b1_ref[...]
    t0 = dis * (s_ref[0] + y_ref[0]) + b1[:, :HID // 2]
    t1 = dis * (s_ref[1] + y_ref[1]) + b1[:, HID // 2:]
    h = jnp.maximum(jnp.concatenate([t0, t1], axis=1), 0.0)
    y2 = jnp.dot(h, w2_ref[...], preferred_element_type=jnp.float32) * dis
    o_ref[0] = y2[:, :OUT // 2]
    o_ref[1] = y2[:, OUT // 2:]


def _tc_f_body(s_ref, y_ref, deg_ref, b2_ref, o_ref):
    dis = _dis_block(deg_ref)
    full = jnp.concatenate(
        [s_ref[0] + y_ref[0], s_ref[1] + y_ref[1]], axis=1)
    o_ref[...] = dis * full + b2_ref[...]


def _halves_spec(w):
    return pl.BlockSpec((NC, ROW_BLK, w), lambda i: (0, i, 0))


_DEG_SPEC = pl.BlockSpec((NC, ROW_BLK, EMBP), lambda i: (0, i, 0))
_GRID = (NP // ROW_BLK,)


def _tc_b(xpad, w1p, deg):
    return pl.pallas_call(
        _tc_b_body,
        grid=_GRID,
        in_specs=[
            pl.BlockSpec((ROW_BLK, INP), lambda i: (i, 0)),
            pl.BlockSpec((INP, HID), lambda i: (0, 0)),
            _DEG_SPEC,
        ],
        out_specs=_halves_spec(HID // 2),
        out_shape=jax.ShapeDtypeStruct((NC, NP, HID // 2), jnp.float32),
    )(xpad, w1p, deg)


def _tc_d(s1h, y1h, deg, b1, w2):
    return pl.pallas_call(
        _tc_d_body,
        grid=_GRID,
        in_specs=[
            _halves_spec(HID // 2),
            _halves_spec(HID // 2),
            _DEG_SPEC,
            pl.BlockSpec((1, HID), lambda i: (0, 0)),
            pl.BlockSpec((HID, OUT), lambda i: (0, 0)),
        ],
        out_specs=_halves_spec(OUT // 2),
        out_shape=jax.ShapeDtypeStruct((NC, NP, OUT // 2), jnp.float32),
    )(s1h, y1h, deg, b1, w2)


def _tc_f(s2h, y2h, deg, b2):
    return pl.pallas_call(
        _tc_f_body,
        grid=_GRID,
        in_specs=[
            _halves_spec(OUT // 2),
            _halves_spec(OUT // 2),
            _DEG_SPEC,
            pl.BlockSpec((1, OUT), lambda i: (0, 0)),
        ],
        out_specs=pl.BlockSpec((ROW_BLK, OUT), lambda i: (i, 0)),
        out_shape=jax.ShapeDtypeStruct((N, OUT), jnp.float32),
    )(s2h, y2h, deg, b2)


def kernel(x_cat, edge_index, emb_tables, W1, b1, W2, b2):
    xcat_pad = jnp.pad(x_cat, ((0, NP - N), (0, 0)))
    fidx = (xcat_pad
            + jnp.arange(NF, dtype=jnp.int32)[None, :] * NCAT).reshape(-1)
    emb_pad = jnp.pad(emb_tables.reshape(NF * NCAT, EMB),
                      ((0, 0), (0, EMBP - EMB)))
    w1p = jnp.zeros((NF, EMBP, HID), jnp.float32).at[:, :EMB, :].set(
        W1.reshape(NF, EMB, HID)).reshape(INP, HID)
    pad_ids = (N + jnp.arange(EP - E, dtype=jnp.int32) % (NP - N))[None, :]
    ei = jnp.concatenate(
        [edge_index, jnp.broadcast_to(pad_ids, (2, EP - E))], axis=1)
    erow = ei[0].reshape(EP // EDGE_CHUNK, EDGE_CHUNK)
    ecol = ei[1].reshape(EP // EDGE_CHUNK, EDGE_CHUNK)
    fidx = fidx.reshape((NP * NF) // EMB_CHUNK, EMB_CHUNK)

    xpad, deg = _emb_deg_kernel(fidx, emb_pad, ecol)
    xpad = xpad.reshape(NP, INP)

    y1h = _tc_b(xpad, w1p, deg)
    s1h = _make_scatter_kernel(HID // 2)(
        y1h.reshape(NC * NP, HID // 2), erow, ecol)
    y2h = _tc_d(s1h, y1h, deg, b1.reshape(1, HID), W2)
    s2h = _make_scatter_kernel(OUT // 2)(
        y2h.reshape(NC * NP, OUT // 2), erow, ecol)
    return _tc_f(s2h, y2h, deg, b2.reshape(1, OUT))


# speedup vs baseline: 1.7486x; 1.0485x over previous
import functools

import jax
import jax.numpy as jnp
from jax import lax
from jax.experimental import pallas as pl
from jax.experimental.pallas import tpu as pltpu
from jax.experimental.pallas import tpu_sc as plsc

N = 10000
NP = 10240
E = 320000
EP = 327680
NF = 16
NCAT = 1000
EMB = 8
DEGW = 16
IN = NF * EMB
HID = 128
OUT = 64

NC = 2
NS = 16
NW = NC * NS
LANES = 16

ROWS_PER_TILE = NP // NS

_MESH = plsc.VectorSubcoreMesh(
    core_axis_name="c", subcore_axis_name="s", num_cores=NC, num_subcores=NS)
_SC_PARAMS = pltpu.CompilerParams(use_tc_tiling_on_sc=False)


def _fill(ref, nrows, ncols, value):
    nv = ncols // LANES
    v = jnp.full((LANES,), value, jnp.float32)

    def body(i, carry):
        r = i // nv
        k = i % nv
        ref[r, pl.ds(k * LANES, LANES)] = v
        return carry

    lax.fori_loop(0, nrows * nv, body, 0)


EMB_CHUNK = 128
EMB_CHUNKS = (NP * NF // NW) // EMB_CHUNK
DEG_CHUNK = 128
DEG_CHUNKS = (EP // NW) // DEG_CHUNK


def _emb_deg_body(fidx_hbm, emb_hbm, ecol_hbm, xpad_out, deg_out,
                  zb16, ones_v, colidx, fidxb, g0, g1, dacc, sem0, sem1,
                  ps0, ps1):
    c = lax.axis_index("c")
    s = lax.axis_index("s")
    w = c * NS + s

    _fill(zb16, ROWS_PER_TILE, DEGW, 0.0)
    pltpu.sync_copy(zb16, dacc.at[pl.ds(s * ROWS_PER_TILE, ROWS_PER_TILE)])
    _fill(ones_v, DEG_CHUNK, DEGW, 1.0)
    pltpu.sync_copy(ecol_hbm.at[pl.ds(w * DEG_CHUNKS, DEG_CHUNKS)], colidx)
    pltpu.sync_copy(fidx_hbm.at[pl.ds(w * EMB_CHUNKS, EMB_CHUNKS)], fidxb)
    plsc.subcore_barrier()

    def dbody(i2, carry):
        i = 2 * i2
        pltpu.async_copy(ones_v, dacc.at[colidx.at[i]], ps0, add=True)
        pltpu.async_copy(ones_v, dacc.at[colidx.at[i + 1]], ps1, add=True)
        pltpu.make_async_copy(ones_v, dacc.at[colidx.at[0]], ps0).wait()
        pltpu.make_async_copy(ones_v, dacc.at[colidx.at[0]], ps1).wait()
        return carry

    lax.fori_loop(0, DEG_CHUNKS // 2, dbody, 0)

    lbase = w * (NP * NF // NW)

    def _fire(i, gb, sm):
        pltpu.async_copy(emb_hbm.at[fidxb.at[i]], gb, sm)

    def _gwait(gb, sm):
        pltpu.make_async_copy(emb_hbm.at[fidxb.at[0]], gb, sm).wait()

    def _put(i, gb, sm):
        pltpu.async_copy(
            gb, xpad_out.at[pl.ds(lbase + i * EMB_CHUNK, EMB_CHUNK)], sm)

    def _pwait(gb, sm):
        pltpu.make_async_copy(
            gb, xpad_out.at[pl.ds(lbase, EMB_CHUNK)], sm).wait()

    _fire(0, g0, sem0)
    _fire(1, g1, sem1)

    def ebody(i2, carry):
        i = 2 * i2
        _gwait(g0, sem0)
        _put(i, g0, ps0)
        _gwait(g1, sem1)
        _put(i + 1, g1, ps1)

        @pl.when(i + 2 < EMB_CHUNKS)
        def _():
            _pwait(g0, ps0)
            _fire(i + 2, g0, sem0)

        @pl.when(i + 3 < EMB_CHUNKS)
        def _():
            _pwait(g1, ps1)
            _fire(i + 3, g1, sem1)

        return carry

    lax.fori_loop(0, EMB_CHUNKS // 2, ebody, 0)
    _pwait(g0, ps0)
    _pwait(g1, ps1)

    plsc.subcore_barrier()
    pltpu.sync_copy(dacc.at[pl.ds(s * ROWS_PER_TILE, ROWS_PER_TILE)], zb16)
    pltpu.sync_copy(zb16, deg_out.at[c, pl.ds(s * ROWS_PER_TILE, ROWS_PER_TILE)])


_emb_deg_kernel = pl.kernel(
    _emb_deg_body,
    out_type=(
        jax.ShapeDtypeStruct((NP * NF, EMB), jnp.float32),
        jax.ShapeDtypeStruct((NC, NP, DEGW), jnp.float32),
    ),
    mesh=_MESH,
    compiler_params=_SC_PARAMS,
    scratch_types=[
        pltpu.VMEM((ROWS_PER_TILE, DEGW), jnp.float32),
        pltpu.VMEM((DEG_CHUNK, DEGW), jnp.float32),
        pltpu.VMEM((DEG_CHUNKS, DEG_CHUNK), jnp.int32),
        pltpu.VMEM((EMB_CHUNKS, EMB_CHUNK), jnp.int32),
        pltpu.VMEM((EMB_CHUNK, EMB), jnp.float32),
        pltpu.VMEM((EMB_CHUNK, EMB), jnp.float32),
        pltpu.VMEM_SHARED((NP, DEGW), jnp.float32),
        pltpu.SemaphoreType.DMA,
        pltpu.SemaphoreType.DMA,
        pltpu.SemaphoreType.DMA,
        pltpu.SemaphoreType.DMA,
    ],
)


EDGE_CHUNK = 128
EDGE_CHUNKS = (EP // NS) // EDGE_CHUNK


def _scatter_body(w, yh_hbm, erow_hbm, ecol_hbm, sh_out,
                  rowidx, colidx, g0, g1, acc, sem0, sem1):
    c = lax.axis_index("c")
    s = lax.axis_index("s")

    _fill(g0, EDGE_CHUNK, w, 0.0)

    def zbody(r, carry):
        pltpu.sync_copy(
            g0, acc.at[pl.ds(s * ROWS_PER_TILE + r * EDGE_CHUNK, EDGE_CHUNK)])
        return carry

    lax.fori_loop(0, ROWS_PER_TILE // EDGE_CHUNK, zbody, 0)
    pltpu.sync_copy(erow_hbm.at[pl.ds(s * EDGE_CHUNKS, EDGE_CHUNKS)], rowidx)
    pltpu.sync_copy(ecol_hbm.at[pl.ds(s * EDGE_CHUNKS, EDGE_CHUNKS)], colidx)
    roff = c * NP

    def adj(k, carry):
        r = k // (EDGE_CHUNK // LANES)
        j = k % (EDGE_CHUNK // LANES)
        rowidx[r, pl.ds(j * LANES, LANES)] = (
            rowidx[r, pl.ds(j * LANES, LANES)] + roff)
        return carry

    @pl.when(c > 0)
    def _():
        lax.fori_loop(0, EDGE_CHUNKS * (EDGE_CHUNK // LANES), adj, 0)

    plsc.subcore_barrier()

    def _fire(i, gb, sm):
        pltpu.async_copy(yh_hbm.at[rowidx.at[i]], gb, sm)

    def _gwait(gb, sm):
        pltpu.make_async_copy(yh_hbm.at[rowidx.at[0]], gb, sm).wait()

    def _scat(i, gb):
        pltpu.sync_copy(gb, acc.at[colidx.at[i]], add=True)

    _fire(0, g0, sem0)

    def body(i2, carry):
        i = 2 * i2
        _fire(i + 1, g1, sem1)
        _gwait(g0, sem0)
        _scat(i, g0)

        @pl.when(i + 2 < EDGE_CHUNKS)
        def _():
            _fire(i + 2, g0, sem0)

        _gwait(g1, sem1)
        _scat(i + 1, g1)
        return carry

    lax.fori_loop(0, EDGE_CHUNKS // 2, body, 0)

    plsc.subcore_barrier()

    def obody(r, carry):
        base = s * ROWS_PER_TILE + r * EDGE_CHUNK
        pltpu.sync_copy(acc.at[pl.ds(base, EDGE_CHUNK)], g0)
        pltpu.sync_copy(g0, sh_out.at[c, pl.ds(base, EDGE_CHUNK)])
        return carry

    lax.fori_loop(0, ROWS_PER_TILE // EDGE_CHUNK, obody, 0)


@functools.cache
def _make_scatter_kernel(w):
    return pl.kernel(
        functools.partial(_scatter_body, w),
        out_type=jax.ShapeDtypeStruct((NC, NP, w), jnp.float32),
        mesh=_MESH,
        compiler_params=_SC_PARAMS,
        scratch_types=[
            pltpu.VMEM((EDGE_CHUNKS, EDGE_CHUNK), jnp.int32),
            pltpu.VMEM((EDGE_CHUNKS, EDGE_CHUNK), jnp.int32),
            pltpu.VMEM((EDGE_CHUNK, w), jnp.float32),
            pltpu.VMEM((EDGE_CHUNK, w), jnp.float32),
            pltpu.VMEM_SHARED((NP, w), jnp.float32),
            pltpu.SemaphoreType.DMA,
            pltpu.SemaphoreType.DMA,
        ],
    )


ROW_BLK = 2048


def _dis_block(deg_ref):
    cnt = deg_ref[0, :, :1] + deg_ref[1, :, :1]
    return lax.rsqrt(cnt + 1.0)


def _tc_b_body(x_ref, w_ref, deg_ref, y_ref):
    dis = _dis_block(deg_ref)
    y = jnp.dot(x_ref[...], w_ref[...], preferred_element_type=jnp.float32)
    y = y * dis
    y_ref[0] = y[:, :HID // 2]
    y_ref[1] = y[:, HID // 2:]


def _tc_d_body(s_ref, y_ref, deg_ref, b1_ref, w2_ref, o_ref):
    dis = _dis_block(deg_ref)
    b1 = b1_ref[...]
    t0 = dis * (s_ref[0] + y_ref[0]) + b1[:, :HID // 2]
    t1 = dis * (s_ref[1] + y_ref[1]) + b1[:, HID // 2:]
    h = jnp.maximum(jnp.concatenate([t0, t1], axis=1), 0.0)
    y2 = jnp.dot(h, w2_ref[...], preferred_element_type=jnp.float32) * dis
    o_ref[0] = y2[:, :OUT // 2]
    o_ref[1] = y2[:, OUT // 2:]


def _tc_f_body(s_ref, y_ref, deg_ref, b2_ref, o_ref):
    dis = _dis_block(deg_ref)
    full = jnp.concatenate(
        [s_ref[0] + y_ref[0], s_ref[1] + y_ref[1]], axis=1)
    o_ref[...] = dis * full + b2_ref[...]


def _halves_spec(w):
    return pl.BlockSpec((NC, ROW_BLK, w), lambda i: (0, i, 0))


_DEG_SPEC = pl.BlockSpec((NC, ROW_BLK, DEGW), lambda i: (0, i, 0))
_GRID = (NP // ROW_BLK,)


def _tc_b(xpad, w1p, deg):
    return pl.pallas_call(
        _tc_b_body,
        grid=_GRID,
        in_specs=[
            pl.BlockSpec((ROW_BLK, IN), lambda i: (i, 0)),
            pl.BlockSpec((IN, HID), lambda i: (0, 0)),
            _DEG_SPEC,
        ],
        out_specs=_halves_spec(HID // 2),
        out_shape=jax.ShapeDtypeStruct((NC, NP, HID // 2), jnp.float32),
    )(xpad, w1p, deg)


def _tc_d(s1h, y1h, deg, b1, w2):
    return pl.pallas_call(
        _tc_d_body,
        grid=_GRID,
        in_specs=[
            _halves_spec(HID // 2),
            _halves_spec(HID // 2),
            _DEG_SPEC,
            pl.BlockSpec((1, HID), lambda i: (0, 0)),
            pl.BlockSpec((HID, OUT), lambda i: (0, 0)),
        ],
        out_specs=_halves_spec(OUT // 2),
        out_shape=jax.ShapeDtypeStruct((NC, NP, OUT // 2), jnp.float32),
    )(s1h, y1h, deg, b1, w2)


def _tc_f(s2h, y2h, deg, b2):
    return pl.pallas_call(
        _tc_f_body,
        grid=_GRID,
        in_specs=[
            _halves_spec(OUT // 2),
            _halves_spec(OUT // 2),
            _DEG_SPEC,
            pl.BlockSpec((1, OUT), lambda i: (0, 0)),
        ],
        out_specs=pl.BlockSpec((ROW_BLK, OUT), lambda i: (i, 0)),
        out_shape=jax.ShapeDtypeStruct((N, OUT), jnp.float32),
    )(s2h, y2h, deg, b2)


def kernel(x_cat, edge_index, emb_tables, W1, b1, W2, b2):
    xcat_pad = jnp.pad(x_cat, ((0, NP - N), (0, 0)))
    fidx = (xcat_pad
            + jnp.arange(NF, dtype=jnp.int32)[None, :] * NCAT).reshape(-1)
    emb_flat = emb_tables.reshape(NF * NCAT, EMB)
    pad_ids = (N + jnp.arange(EP - E, dtype=jnp.int32) % (NP - N))[None, :]
    ei = jnp.concatenate(
        [edge_index, jnp.broadcast_to(pad_ids, (2, EP - E))], axis=1)
    erow = ei[0].reshape(EP // EDGE_CHUNK, EDGE_CHUNK)
    ecol = ei[1].reshape(EP // EDGE_CHUNK, EDGE_CHUNK)
    fidx = fidx.reshape((NP * NF) // EMB_CHUNK, EMB_CHUNK)

    xpad, deg = _emb_deg_kernel(fidx, emb_flat, ecol)
    xpad = xpad.reshape(NP, IN)

    y1h = _tc_b(xpad, W1, deg)
    s1h = _make_scatter_kernel(HID // 2)(
        y1h.reshape(NC * NP, HID // 2), erow, ecol)
    y2h = _tc_d(s1h, y1h, deg, b1.reshape(1, HID), W2)
    s2h = _make_scatter_kernel(OUT // 2)(
        y2h.reshape(NC * NP, OUT // 2), erow, ecol)
    return _tc_f(s2h, y2h, deg, b2.reshape(1, OUT))
